# Initial kernel scaffold; baseline (speedup 1.0000x reference)
import jax
import jax.numpy as jnp
from jax.experimental import pallas as pl


def kernel(centers, pred_offset, dims_offset, depths_offset, vector_ori, cls_id):
    raise NotImplementedError("write your pallas kernel here")


# speedup vs baseline: 1.2294x; 1.2294x over previous
import functools

import jax
import jax.numpy as jnp
import numpy as np
from jax import lax
from jax.experimental import pallas as pl
from jax.experimental.pallas import tpu as pltpu
from jax.experimental.pallas import tpu_sc as plsc

PI = float(np.pi)
DOWN_RATIO = 4.0
INPUT_W = 1280.0
INPUT_H = 384.0
DEPTH_MIN, DEPTH_MAX = 0.1, 100.0

_DIM_MEAN = ((4.83899871, 1.80778956, 2.11565798),
             (0.91986743, 1.75302337, 0.86220807),
             (1.78652745, 1.76500989, 0.83395625))

_ATAN_C = (0.9998660, -0.3302995, 0.1801410, -0.0851330, 0.0208351)

_L = 16
_NW = 32


def _atan_poly(r):
    a = jnp.abs(r)
    big = a > 1.0
    t = jnp.where(big, 1.0 / a, a)
    t2 = t * t
    p = jnp.full_like(t, _ATAN_C[4])
    for c in (_ATAN_C[3], _ATAN_C[2], _ATAN_C[1], _ATAN_C[0]):
        p = p * t2 + c
    p = p * t
    p = jnp.where(big, (PI / 2) - p, p)
    return jnp.where(r < 0.0, -p, p)


@functools.cache
def _build(N: int):
    f32 = jnp.float32
    i32 = jnp.int32
    G = N // _L
    GPW = -(-G // _NW)
    ROWS = GPW * _L
    mesh = plsc.VectorSubcoreMesh(core_axis_name="c", subcore_axis_name="s",
                                  num_cores=2, num_subcores=16)

    out_type = (jax.ShapeDtypeStruct((N * 4,), f32),
                jax.ShapeDtypeStruct((N * 3,), f32),
                jax.ShapeDtypeStruct((N,), f32),
                jax.ShapeDtypeStruct((N,), f32))
    scratch = [
        pltpu.VMEM((ROWS * 2,), f32),
        pltpu.VMEM((ROWS * 4,), f32),
        pltpu.VMEM((ROWS * 3,), f32),
        pltpu.VMEM((ROWS,), f32),
        pltpu.VMEM((ROWS * 16,), f32),
        pltpu.VMEM((ROWS,), i32),
        pltpu.VMEM((ROWS * 4,), f32),
        pltpu.VMEM((ROWS * 3,), f32),
        pltpu.VMEM((ROWS,), f32),
        pltpu.VMEM((ROWS,), f32),
    ]

    @functools.partial(
        pl.kernel, out_type=out_type, mesh=mesh, scratch_types=scratch,
        compiler_params=pltpu.CompilerParams(needs_layout_passes=False))
    def _k(cen_h, po_h, do_h, dep_h, vo_h, cls_h,
           bo_h, dm_h, dp_h, al_h,
           cen_v, po_v, do_v, dep_v, vo_v, cls_v,
           bo_v, dm_v, dp_v, al_v):
        w = lax.axis_index("s") * 2 + lax.axis_index("c")
        base = jnp.minimum(w * ROWS, N - ROWS)
        pltpu.sync_copy(cen_h.at[pl.ds(base * 2, ROWS * 2)], cen_v)
        pltpu.sync_copy(po_h.at[pl.ds(base * 4, ROWS * 4)], po_v)
        pltpu.sync_copy(do_h.at[pl.ds(base * 3, ROWS * 3)], do_v)
        pltpu.sync_copy(dep_h.at[pl.ds(base, ROWS)], dep_v)
        pltpu.sync_copy(vo_h.at[pl.ds(base * 16, ROWS * 16)], vo_v)
        pltpu.sync_copy(cls_h.at[pl.ds(base, ROWS)], cls_v)

        iota = lax.iota(i32, _L)

        def group(g, carry):
            r0 = g * _L
            rows = r0 + iota
            rows2 = rows * 2
            rows3 = rows * 3
            rows4 = rows * 4
            rows16 = rows * 16

            cx = plsc.load_gather(cen_v, [rows2])
            cy = plsc.load_gather(cen_v, [rows2 + 1])
            x1 = (cx - plsc.load_gather(po_v, [rows4])) * DOWN_RATIO
            y1 = (cy - plsc.load_gather(po_v, [rows4 + 1])) * DOWN_RATIO
            x2 = (cx + plsc.load_gather(po_v, [rows4 + 2])) * DOWN_RATIO
            y2 = (cy + plsc.load_gather(po_v, [rows4 + 3])) * DOWN_RATIO
            plsc.store_scatter(bo_v, [rows4], jnp.clip(x1, 0.0, INPUT_W))
            plsc.store_scatter(bo_v, [rows4 + 1], jnp.clip(y1, 0.0, INPUT_H))
            plsc.store_scatter(bo_v, [rows4 + 2], x2)
            plsc.store_scatter(bo_v, [rows4 + 3], y2)

            cls16 = cls_v[pl.ds(r0, _L)]
            is0 = cls16 == 0
            is1 = cls16 == 1
            for j in range(3):
                mj = jnp.where(is0, _DIM_MEAN[0][j],
                               jnp.where(is1, _DIM_MEAN[1][j],
                                         _DIM_MEAN[2][j]))
                dj = plsc.load_gather(do_v, [rows3 + j])
                plsc.store_scatter(dm_v, [rows3 + j], jnp.exp(dj) * mj)

            dp_v[pl.ds(r0, _L)] = jnp.clip(
                jnp.exp(-dep_v[pl.ds(r0, _L)]), DEPTH_MIN, DEPTH_MAX)

            m = (plsc.load_gather(vo_v, [rows16 + 1])
                 - plsc.load_gather(vo_v, [rows16]))
            best = jnp.zeros((_L,), i32)
            for k in (1, 2, 3):
                dk = (plsc.load_gather(vo_v, [rows16 + (2 * k + 1)])
                      - plsc.load_gather(vo_v, [rows16 + 2 * k]))
                gt = dk > m
                m = jnp.where(gt, dk, m)
                best = jnp.where(gt, k, best)
            sel0 = rows16 + 8 + 2 * best
            s0 = plsc.load_gather(vo_v, [sel0])
            s1 = plsc.load_gather(vo_v, [sel0 + 1])
            alpha = _atan_poly(s0 / s1)
            alpha = alpha + jnp.where(best == 3, -(PI / 2),
                                      best.astype(f32) * (PI / 2))
            alpha = jnp.where(alpha > PI, alpha - 2 * PI, alpha)
            alpha = jnp.where(alpha < -PI, alpha + 2 * PI, alpha)
            al_v[pl.ds(r0, _L)] = alpha
            return carry

        lax.fori_loop(0, GPW, group, 0)

        pltpu.sync_copy(bo_v, bo_h.at[pl.ds(base * 4, ROWS * 4)])
        pltpu.sync_copy(dm_v, dm_h.at[pl.ds(base * 3, ROWS * 3)])
        pltpu.sync_copy(dp_v, dp_h.at[pl.ds(base, ROWS)])
        pltpu.sync_copy(al_v, al_h.at[pl.ds(base, ROWS)])

    return _k


def kernel(centers, pred_offset, dims_offset, depths_offset, vector_ori,
           cls_id):
    N = centers.shape[0]
    k = _build(N)
    box, dims, depth, alphas = k(
        centers.astype(jnp.float32).reshape(-1),
        pred_offset.astype(jnp.float32).reshape(-1),
        dims_offset.astype(jnp.float32).reshape(-1),
        depths_offset.astype(jnp.float32).reshape(-1),
        vector_ori.astype(jnp.float32).reshape(-1),
        cls_id.astype(jnp.int32))
    return (box.reshape(N, 4), dims.reshape(N, 3), depth, alphas)
